# baseline (speedup 1.0000x reference)
import functools

import jax
import jax.numpy as jnp
import numpy as np
from jax.experimental import pallas as pl
from jax.experimental.pallas import tpu as pltpu

_N_BINS = 32
_R = 1.2
_BOUND = 50.0
_N_LENGTH = 16


def _np_mesh_constants():
    m = _N_BINS / 2
    x1L = _BOUND * (_R - 1.0) / (_R**m - 1.0)
    index = np.arange(0, _N_BINS + 1, dtype=np.float64).reshape(-1, 1) - m
    xr = np.where(index >= 0,
                  (1.0 - _R**index) / (1.0 - _R),
                  (1.0 - _R**np.abs(index)) / (1.0 - _R))
    xr = np.where(index >= 0, x1L * xr, -x1L * xr)
    xr = (xr + _BOUND) / 2.0 / _BOUND
    x1L_s = x1L / 2.0 / _BOUND
    mesh = np.concatenate([np.zeros((1, 1)), xr[1:-1, 0:1], np.ones((1, 1))], 0)
    elmt = (mesh[1:] - mesh[:-1]).reshape(-1, 1)
    return (mesh.astype(np.float32), elmt.astype(np.float32),
            np.float32(x1L_s))


_MESH, _ELMT, _X1L = _np_mesh_constants()
_ELMT_ROW = _ELMT.reshape(1, _N_BINS)
_W_ROW = ((_ELMT[:-1, 0] + _ELMT[1:, 0]) / 2.0).reshape(1, _N_BINS - 1)
_MESH_ROW = _MESH[:_N_BINS, 0].reshape(1, _N_BINS)
_TRIU = (np.arange(_N_BINS)[:, None] < np.arange(_N_BINS)[None, :]).astype(
    np.float32)
_ACOEF = float((_R - 1.0) / _X1L)
_INV_LOG_R = float(1.0 / np.log(_R))


def _prep_kernel(p_ref, elmt_ref, w_ref, mesh_ref, triu_ref, t_ref):
    pt = p_ref[...]
    ep = jnp.exp(pt)
    s = jnp.sum(ep * w_ref[...], axis=1, keepdims=True)
    px = ((1.0 - float(_ELMT[0, 0])) / s) * ep
    one = jnp.ones((_N_LENGTH, 1), jnp.float32)
    pdf = jnp.concatenate([one, px, one], axis=1)
    elmt = elmt_ref[...]
    cell = (pdf[:, :-1] + pdf[:, 1:]) / 2.0 * elmt
    f_ref = jnp.dot(cell, triu_ref[...],
                    preferred_element_type=jnp.float32,
                    precision=jax.lax.Precision.HIGHEST)
    a = f_ref * (2.0 * _BOUND) - _BOUND
    b = pdf[:, :_N_BINS] * (2.0 * _BOUND)
    c = (pdf[:, 1:] - pdf[:, :-1]) / (2.0 * elmt) * (2.0 * _BOUND)
    ub = jax.lax.bitcast_convert_type(b, jnp.uint32)
    uc = jax.lax.bitcast_convert_type(c, jnp.uint32)
    ub = (ub + 0x8000) & jnp.uint32(0xFFFF0000)
    uc = (uc + 0x8000) >> 16
    bc = jax.lax.bitcast_convert_type(ub | uc, jnp.float32)
    t_ref[...] = jnp.concatenate([a, bc], axis=1)


_LOG2R = float(np.log2(_R))
_INV_A = float(1.0 / _ACOEF)


def _main_kernel(t_ref, x_ref, o_ref):
    t = t_ref[...]
    x = x_ref[...]
    d = x * (0.5 / _BOUND)
    ad = jnp.abs(d)
    tt = ad * _ACOEF + 1.0
    pos = d >= 0
    mf = jnp.trunc(jnp.log(tt) * _INV_LOG_R)
    km = mf.astype(jnp.int32)
    ki = 16 + jnp.where(pos, km, ~km)
    inr = (ki & ~31) == 0
    kg = ki & 31

    def gather(off):
        return jnp.take_along_axis(t, kg + off, axis=1,
                                   mode="promise_in_bounds")

    a = gather(0)
    gbc = jax.lax.bitcast_convert_type(gather(32), jnp.uint32)
    b = jax.lax.bitcast_convert_type(gbc & jnp.uint32(0xFFFF0000),
                                     jnp.float32)
    c = jax.lax.bitcast_convert_type(gbc << 16, jnp.float32)
    mprime = jnp.where(pos, mf, mf + 1.0)
    g = (jnp.exp2(mprime * _LOG2R) - 1.0) * _INV_A
    axm = ad - g
    xm = jnp.where(pos, axm, -axm)
    y = a + xm * (b + xm * c)
    o_ref[...] = jnp.where(inr, y, x)


_W = 65536


@functools.partial(jax.jit, static_argnames=("interpret",))
def kernel(x, p, interpret=False):
    batch, n_len = x.shape
    xt = x.T
    pt = p.T

    t = pl.pallas_call(
        _prep_kernel,
        out_shape=jax.ShapeDtypeStruct((_N_LENGTH, 64), jnp.float32),
        interpret=interpret,
    )(pt, jnp.asarray(_ELMT_ROW), jnp.asarray(_W_ROW),
      jnp.asarray(_MESH_ROW), jnp.asarray(_TRIU))

    grid = batch // _W
    yt = pl.pallas_call(
        _main_kernel,
        grid=(grid,),
        in_specs=[
            pl.BlockSpec((_N_LENGTH, 64), lambda i: (0, 0)),
            pl.BlockSpec((_N_LENGTH, _W), lambda i: (0, i)),
        ],
        out_specs=pl.BlockSpec((_N_LENGTH, _W), lambda i: (0, i)),
        out_shape=jax.ShapeDtypeStruct((n_len, batch), jnp.float32),
        compiler_params=pltpu.CompilerParams(
            dimension_semantics=("parallel",),
        ),
        interpret=interpret,
    )(t, xt)
    return yt.T


# speedup vs baseline: 1.0016x; 1.0016x over previous
import functools

import jax
import jax.numpy as jnp
import numpy as np
from jax.experimental import pallas as pl
from jax.experimental.pallas import tpu as pltpu

_N_BINS = 32
_R = 1.2
_BOUND = 50.0
_N_LENGTH = 16


def _np_mesh_constants():
    m = _N_BINS / 2
    x1L = _BOUND * (_R - 1.0) / (_R**m - 1.0)
    index = np.arange(0, _N_BINS + 1, dtype=np.float64).reshape(-1, 1) - m
    xr = np.where(index >= 0,
                  (1.0 - _R**index) / (1.0 - _R),
                  (1.0 - _R**np.abs(index)) / (1.0 - _R))
    xr = np.where(index >= 0, x1L * xr, -x1L * xr)
    xr = (xr + _BOUND) / 2.0 / _BOUND
    x1L_s = x1L / 2.0 / _BOUND
    mesh = np.concatenate([np.zeros((1, 1)), xr[1:-1, 0:1], np.ones((1, 1))], 0)
    elmt = (mesh[1:] - mesh[:-1]).reshape(-1, 1)
    return (mesh.astype(np.float32), elmt.astype(np.float32),
            np.float32(x1L_s))


_MESH, _ELMT, _X1L = _np_mesh_constants()
_ELMT_ROW = _ELMT.reshape(1, _N_BINS)
_W_ROW = ((_ELMT[:-1, 0] + _ELMT[1:, 0]) / 2.0).reshape(1, _N_BINS - 1)
_MESH_ROW = _MESH[:_N_BINS, 0].reshape(1, _N_BINS)
_TRIU = (np.arange(_N_BINS)[:, None] < np.arange(_N_BINS)[None, :]).astype(
    np.float32)
_ACOEF = float((_R - 1.0) / _X1L)
_INV_LOG_R = float(1.0 / np.log(_R))


def _prep_kernel(p_ref, elmt_ref, w_ref, mesh_ref, triu_ref, t_ref):
    pt = p_ref[...]
    ep = jnp.exp(pt)
    s = jnp.sum(ep * w_ref[...], axis=1, keepdims=True)
    px = ((1.0 - float(_ELMT[0, 0])) / s) * ep
    one = jnp.ones((_N_LENGTH, 1), jnp.float32)
    pdf = jnp.concatenate([one, px, one], axis=1)
    elmt = elmt_ref[...]
    cell = (pdf[:, :-1] + pdf[:, 1:]) / 2.0 * elmt
    f_ref = jnp.dot(cell, triu_ref[...],
                    preferred_element_type=jnp.float32,
                    precision=jax.lax.Precision.HIGHEST)
    a = f_ref * (2.0 * _BOUND) - _BOUND
    b = pdf[:, :_N_BINS] * (2.0 * _BOUND)
    c = (pdf[:, 1:] - pdf[:, :-1]) / (2.0 * elmt) * (2.0 * _BOUND)
    ub = jax.lax.bitcast_convert_type(b, jnp.uint32)
    uc = jax.lax.bitcast_convert_type(c, jnp.uint32)
    ub = (ub + 0x8000) & jnp.uint32(0xFFFF0000)
    uc = (uc + 0x8000) >> 16
    bc = jax.lax.bitcast_convert_type(ub | uc, jnp.float32)
    t_ref[...] = jnp.concatenate([a, bc], axis=1)


_LOG2R = float(np.log2(_R))
_INV_A = float(1.0 / _ACOEF)


def _main_kernel(t_ref, x_ref, o_ref):
    t = t_ref[...]
    x = x_ref[...]
    d = x * (0.5 / _BOUND)
    ad = jnp.abs(d)
    tt = ad * _ACOEF + 1.0
    mf = jnp.trunc(jnp.log(tt) * _INV_LOG_R)
    km = mf.astype(jnp.int32)
    pos = d >= 0
    ki = 16 + jnp.where(pos, km, ~km)
    inr = (ki & ~31) == 0
    kg = ki & 31

    def gather(off):
        return jnp.take_along_axis(t, kg + off, axis=1,
                                   mode="promise_in_bounds")

    a = gather(0)
    gbc = jax.lax.bitcast_convert_type(gather(32), jnp.uint32)
    b = jax.lax.bitcast_convert_type(gbc & jnp.uint32(0xFFFF0000),
                                     jnp.float32)
    c = jax.lax.bitcast_convert_type(gbc << 16, jnp.float32)
    mprime = jnp.where(pos, mf, mf + 1.0)
    g = (jnp.exp2(mprime * _LOG2R) - 1.0) * _INV_A
    axm = ad - g
    xm = jnp.where(pos, axm, -axm)
    y = a + xm * (b + xm * c)
    o_ref[...] = jnp.where(inr, y, x)


_W = 32768


@functools.partial(jax.jit, static_argnames=("interpret",))
def kernel(x, p, interpret=False):
    batch, n_len = x.shape
    xt = x.T
    pt = p.T

    t = pl.pallas_call(
        _prep_kernel,
        out_shape=jax.ShapeDtypeStruct((_N_LENGTH, 64), jnp.float32),
        interpret=interpret,
    )(pt, jnp.asarray(_ELMT_ROW), jnp.asarray(_W_ROW),
      jnp.asarray(_MESH_ROW), jnp.asarray(_TRIU))

    grid = batch // _W
    yt = pl.pallas_call(
        _main_kernel,
        grid=(grid,),
        in_specs=[
            pl.BlockSpec((_N_LENGTH, 64), lambda i: (0, 0)),
            pl.BlockSpec((_N_LENGTH, _W), lambda i: (0, i)),
        ],
        out_specs=pl.BlockSpec((_N_LENGTH, _W), lambda i: (0, i)),
        out_shape=jax.ShapeDtypeStruct((n_len, batch), jnp.float32),
        compiler_params=pltpu.CompilerParams(
            dimension_semantics=("parallel",),
        ),
        interpret=interpret,
    )(t, xt)
    return yt.T


# speedup vs baseline: 1.0025x; 1.0009x over previous
import jax
import jax.numpy as jnp
import numpy as np
from jax.experimental import pallas as pl
from jax.experimental.pallas import tpu as pltpu

_N_BINS = 32
_R = 1.2
_BOUND = 50.0
_N_LENGTH = 16


def _np_mesh_constants():
    m = _N_BINS / 2
    x1L = _BOUND * (_R - 1.0) / (_R**m - 1.0)
    index = np.arange(0, _N_BINS + 1, dtype=np.float64).reshape(-1, 1) - m
    xr = np.where(index >= 0,
                  (1.0 - _R**index) / (1.0 - _R),
                  (1.0 - _R**np.abs(index)) / (1.0 - _R))
    xr = np.where(index >= 0, x1L * xr, -x1L * xr)
    xr = (xr + _BOUND) / 2.0 / _BOUND
    x1L_s = x1L / 2.0 / _BOUND
    mesh = np.concatenate([np.zeros((1, 1)), xr[1:-1, 0:1], np.ones((1, 1))], 0)
    elmt = (mesh[1:] - mesh[:-1]).reshape(-1, 1)
    return (mesh.astype(np.float32), elmt.astype(np.float32),
            np.float32(x1L_s))


_MESH, _ELMT, _X1L = _np_mesh_constants()
_ELMT_ROW = _ELMT.reshape(1, _N_BINS)
_W_ROW = ((_ELMT[:-1, 0] + _ELMT[1:, 0]) / 2.0).reshape(1, _N_BINS - 1)
_MESH_ROW = _MESH[:_N_BINS, 0].reshape(1, _N_BINS)
_TRIU = (np.arange(_N_BINS)[:, None] < np.arange(_N_BINS)[None, :]).astype(
    np.float32)
_ACOEF = float((_R - 1.0) / _X1L)
_INV_LOG_R = float(1.0 / np.log(_R))


def _prep_kernel(p_ref, elmt_ref, w_ref, mesh_ref, triu_ref, t_ref):
    pt = p_ref[...]
    ep = jnp.exp(pt)
    s = jnp.sum(ep * w_ref[...], axis=1, keepdims=True)
    px = ((1.0 - float(_ELMT[0, 0])) / s) * ep
    one = jnp.ones((_N_LENGTH, 1), jnp.float32)
    pdf = jnp.concatenate([one, px, one], axis=1)
    elmt = elmt_ref[...]
    cell = (pdf[:, :-1] + pdf[:, 1:]) / 2.0 * elmt
    f_ref = jnp.dot(cell, triu_ref[...],
                    preferred_element_type=jnp.float32,
                    precision=jax.lax.Precision.HIGHEST)
    a = f_ref * (2.0 * _BOUND) - _BOUND
    b = pdf[:, :_N_BINS] * (2.0 * _BOUND)
    c = (pdf[:, 1:] - pdf[:, :-1]) / (2.0 * elmt) * (2.0 * _BOUND)
    ub = jax.lax.bitcast_convert_type(b, jnp.uint32)
    uc = jax.lax.bitcast_convert_type(c, jnp.uint32)
    ub = (ub + 0x8000) & jnp.uint32(0xFFFF0000)
    uc = (uc + 0x8000) >> 16
    bc = jax.lax.bitcast_convert_type(ub | uc, jnp.float32)
    t_ref[...] = jnp.concatenate([a, bc], axis=1)


_LOG2R = float(np.log2(_R))
_INV_A = float(1.0 / _ACOEF)


def _main_kernel(t_ref, x_ref, o_ref):
    t = t_ref[...]
    x = x_ref[...]
    d = x * (0.5 / _BOUND)
    ad = jnp.abs(d)
    tt = ad * _ACOEF + 1.0
    mf = jnp.trunc(jnp.log(tt) * _INV_LOG_R)
    km = mf.astype(jnp.int32)
    pos = d >= 0
    ki = 16 + jnp.where(pos, km, ~km)
    inr = (ki & ~31) == 0
    kg = ki & 31

    def gather(off):
        return jnp.take_along_axis(t, kg + off, axis=1,
                                   mode="promise_in_bounds")

    a = gather(0)
    gbc = jax.lax.bitcast_convert_type(gather(32), jnp.uint32)
    b = jax.lax.bitcast_convert_type(gbc & jnp.uint32(0xFFFF0000),
                                     jnp.float32)
    c = jax.lax.bitcast_convert_type(gbc << 16, jnp.float32)
    mprime = jnp.where(pos, mf, mf + 1.0)
    g = (jnp.exp2(mprime * _LOG2R) - 1.0) * _INV_A
    axm = ad - g
    xm = jnp.where(pos, axm, -axm)
    y = a + xm * (b + xm * c)
    o_ref[...] = jnp.where(inr, y, x)


_W = 32768


@jax.jit
def kernel(x, p):
    batch, n_len = x.shape
    xt = x.T
    pt = p.T

    t = pl.pallas_call(
        _prep_kernel,
        out_shape=jax.ShapeDtypeStruct((_N_LENGTH, 64), jnp.float32),
    )(pt, jnp.asarray(_ELMT_ROW), jnp.asarray(_W_ROW),
      jnp.asarray(_MESH_ROW), jnp.asarray(_TRIU))

    grid = batch // _W
    yt = pl.pallas_call(
        _main_kernel,
        grid=(grid,),
        in_specs=[
            pl.BlockSpec((_N_LENGTH, 64), lambda i: (0, 0)),
            pl.BlockSpec((_N_LENGTH, _W), lambda i: (0, i)),
        ],
        out_specs=pl.BlockSpec((_N_LENGTH, _W), lambda i: (0, i)),
        out_shape=jax.ShapeDtypeStruct((n_len, batch), jnp.float32),
        compiler_params=pltpu.CompilerParams(
            dimension_semantics=("parallel",),
        ),
    )(t, xt)
    return yt.T
